# baseline (speedup 1.0000x reference)
import jax
import jax.numpy as jnp
from jax.experimental import pallas as pl


def _linear_kernel(h_ref, w_ref, b_ref, out_ref):
    out_ref[...] = jax.lax.dot_general(
        h_ref[...], w_ref[...],
        dimension_numbers=(((1,), (1,)), ((), ())),
        preferred_element_type=jnp.float32,
    ) + b_ref[...]


def kernel(h, edge_index, W, b):
    n, d_in = h.shape
    d_out = W.shape[0]
    block = 1000
    return pl.pallas_call(
        _linear_kernel,
        grid=(n // block,),
        in_specs=[
            pl.BlockSpec((block, d_in), lambda i: (i, 0)),
            pl.BlockSpec((d_out, d_in), lambda i: (0, 0)),
            pl.BlockSpec((1, d_out), lambda i: (0, 0)),
        ],
        out_specs=pl.BlockSpec((block, d_out), lambda i: (i, 0)),
        out_shape=jax.ShapeDtypeStruct((n, d_out), jnp.float32),
    )(h, W, b.reshape(1, d_out))


# speedup vs baseline: 1.8759x; 1.8759x over previous
import jax
import jax.numpy as jnp
from jax.experimental import pallas as pl


def _linear_kernel(h_ref, w_ref, b_ref, out_ref):
    out_ref[...] = jax.lax.dot_general(
        h_ref[...], w_ref[...],
        dimension_numbers=(((1,), (1,)), ((), ())),
        preferred_element_type=jnp.float32,
    ) + b_ref[...]


def kernel(h, edge_index, W, b):
    n, d_in = h.shape
    d_out = W.shape[0]
    block = 5000
    return pl.pallas_call(
        _linear_kernel,
        grid=(n // block,),
        in_specs=[
            pl.BlockSpec((block, d_in), lambda i: (i, 0)),
            pl.BlockSpec((d_out, d_in), lambda i: (0, 0)),
            pl.BlockSpec((1, d_out), lambda i: (0, 0)),
        ],
        out_specs=pl.BlockSpec((block, d_out), lambda i: (i, 0)),
        out_shape=jax.ShapeDtypeStruct((n, d_out), jnp.float32),
    )(h, W, b.reshape(1, d_out))
